# Initial kernel scaffold; baseline (speedup 1.0000x reference)
import jax
import jax.numpy as jnp
from jax.experimental import pallas as pl


def kernel(x, edge_index, batch, W0, b0, W1, b1, W2, b2, W3, b3, Wout, bout):
    raise NotImplementedError("write your pallas kernel here")


# speedup vs baseline: 13.1381x; 13.1381x over previous
import functools

import jax
import jax.numpy as jnp
from jax import lax
from jax.experimental import pallas as pl
from jax.experimental.pallas import tpu as pltpu
from jax.experimental.pallas import tpu_sc as plsc

N = 10000
E = 320000
H = 128
HD = H // 2
G = 64

NC = 2
NS = 16
CW = 125
CHT = 160
RPT = 624
TAIL = N - NS * RPT
ZB = 208

_mesh = plsc.VectorSubcoreMesh(core_axis_name="c", subcore_axis_name="s")


@functools.partial(
    pl.kernel,
    out_type=jax.ShapeDtypeStruct((NC, N, 16), jnp.float32),
    mesh=_mesh,
    scratch_types=[
        pltpu.VMEM((CHT, CW), jnp.int32),
        pltpu.VMEM((CW, 16), jnp.float32),
        pltpu.VMEM((RPT, 16), jnp.float32),
        pltpu.VMEM_SHARED((N, 16), jnp.float32),
    ],
    compiler_params=pltpu.CompilerParams(use_tc_tiling_on_sc=False),
)
def _sc_deg(er_hbm, out_hbm, dst_v, ones_v, zero_v, acc_sh):
    c = lax.axis_index("c")
    s = lax.axis_index("s")

    @pl.loop(0, CW)
    def _(i):
        ones_v[i, :] = jnp.ones((16,), jnp.float32)

    @pl.loop(0, RPT)
    def _(i):
        zero_v[i, :] = jnp.zeros((16,), jnp.float32)

    pltpu.sync_copy(zero_v, acc_sh.at[pl.ds(s * RPT, RPT)])

    @pl.when(s == NS - 1)
    def _():
        pltpu.sync_copy(zero_v.at[pl.ds(0, TAIL)],
                        acc_sh.at[pl.ds(NS * RPT, TAIL)])

    plsc.subcore_barrier()

    pltpu.sync_copy(er_hbm.at[2, s], dst_v)

    @pl.loop(0, CHT // NC)
    def _(j):
        pltpu.sync_copy(ones_v, acc_sh.at[dst_v.at[c * (CHT // NC) + j]],
                        add=True)

    plsc.subcore_barrier()
    pltpu.sync_copy(acc_sh.at[pl.ds(s * RPT, RPT)],
                    out_hbm.at[c, pl.ds(s * RPT, RPT)])

    @pl.when(s == NS - 1)
    def _():
        pltpu.sync_copy(acc_sh.at[pl.ds(NS * RPT, TAIL)],
                        out_hbm.at[c, pl.ds(NS * RPT, TAIL)])


@functools.partial(
    pl.kernel,
    out_type=jax.ShapeDtypeStruct((NC, N, HD), jnp.float32),
    mesh=_mesh,
    scratch_types=[
        pltpu.VMEM((CHT, CW), jnp.int32),
        pltpu.VMEM((CHT, CW), jnp.int32),
        pltpu.VMEM((CW, HD), jnp.float32),
        pltpu.VMEM((ZB, HD), jnp.float32),
        pltpu.VMEM_SHARED((N, HD), jnp.float32),
        pltpu.SemaphoreType.DMA,
    ],
    compiler_params=pltpu.CompilerParams(use_tc_tiling_on_sc=False),
)
def _sc_agg(u_flat, er_hbm, out_hbm, src_v, dst_v, rows_v, zero_v, acc_sh, sem):
    u_hbm = u_flat
    c = lax.axis_index("c")
    s = lax.axis_index("s")

    @pl.loop(0, ZB)
    def _(i):
        for k in range(HD // 16):
            zero_v[i, pl.ds(16 * k, 16)] = jnp.zeros((16,), jnp.float32)

    for t in range(RPT // ZB):
        pltpu.sync_copy(zero_v, acc_sh.at[pl.ds(s * RPT + t * ZB, ZB)])

    @pl.when(s == NS - 1)
    def _():
        pltpu.sync_copy(zero_v.at[pl.ds(0, TAIL)],
                        acc_sh.at[pl.ds(NS * RPT, TAIL)])

    plsc.subcore_barrier()

    pltpu.sync_copy(er_hbm.at[c, s], src_v)
    pltpu.sync_copy(er_hbm.at[2, s], dst_v)

    @pl.loop(0, CHT)
    def _(j):
        pltpu.async_copy(u_hbm.at[src_v.at[j]], rows_v, sem).wait()
        pltpu.sync_copy(rows_v, acc_sh.at[dst_v.at[j]], add=True)

    plsc.subcore_barrier()
    pltpu.sync_copy(acc_sh.at[pl.ds(s * RPT, RPT)],
                    out_hbm.at[c, pl.ds(s * RPT, RPT)])

    @pl.when(s == NS - 1)
    def _():
        pltpu.sync_copy(acc_sh.at[pl.ds(NS * RPT, TAIL)],
                        out_hbm.at[c, pl.ds(NS * RPT, TAIL)])


_RB = 2000


def _dinv_of(dref):
    deg = dref[0, :, 0:1] + dref[1, :, 0:1] + 1.0
    return lax.rsqrt(deg)


def _split_store(o_ref, t, dinv):
    u = t * dinv
    o_ref[0:1, :, :] = u[None, :, 0:HD]
    o_ref[1:2, :, :] = u[None, :, HD:H]


def _tc_layer0_body(x_ref, d_ref, w_ref, o_ref):
    dinv = _dinv_of(d_ref)
    t = jnp.dot(x_ref[...], w_ref[...], preferred_element_type=jnp.float32)
    _split_store(o_ref, t, dinv)


def _tc_layer0(x, deg2, W0):
    grid = (N // _RB,)
    return pl.pallas_call(
        _tc_layer0_body,
        grid=grid,
        in_specs=[
            pl.BlockSpec((_RB, H), lambda i: (i, 0)),
            pl.BlockSpec((NC, _RB, 16), lambda i: (0, i, 0)),
            pl.BlockSpec((H, H), lambda i: (0, 0)),
        ],
        out_specs=pl.BlockSpec((NC, _RB, HD), lambda i: (0, i, 0)),
        out_shape=jax.ShapeDtypeStruct((NC, N, HD), jnp.float32),
    )(x, deg2, W0)


def _hcat(a_ref):
    return jnp.concatenate([a_ref[0], a_ref[1]], axis=1)


def _tc_layer_body(a_ref, u_ref, d_ref, b_ref, w_ref, o_ref):
    dinv = _dinv_of(d_ref)
    pre = dinv * (_hcat(a_ref) + _hcat(u_ref)) + b_ref[...]
    h = jnp.tanh(pre)
    t = jnp.dot(h, w_ref[...], preferred_element_type=jnp.float32)
    _split_store(o_ref, t, dinv)


def _tc_layer(agg, u, deg2, b, Wnext):
    grid = (N // _RB,)
    return pl.pallas_call(
        _tc_layer_body,
        grid=grid,
        in_specs=[
            pl.BlockSpec((NC, _RB, HD), lambda i: (0, i, 0)),
            pl.BlockSpec((NC, _RB, HD), lambda i: (0, i, 0)),
            pl.BlockSpec((NC, _RB, 16), lambda i: (0, i, 0)),
            pl.BlockSpec((1, H), lambda i: (0, 0)),
            pl.BlockSpec((H, H), lambda i: (0, 0)),
        ],
        out_specs=pl.BlockSpec((NC, _RB, HD), lambda i: (0, i, 0)),
        out_shape=jax.ShapeDtypeStruct((NC, N, HD), jnp.float32),
    )(agg, u, deg2, b.reshape(1, H), Wnext)


_PB = 400
_NPB = N // _PB


def _tc_final_body(a_ref, u_ref, d_ref, b_ref, batch_ref, bcol_ref, w_ref,
                   bo_ref, o_ref, smax, ssum, scnt):
    i = pl.program_id(0)

    @pl.when(i == 0)
    def _():
        smax[...] = jnp.full((G, H), -jnp.inf, jnp.float32)
        ssum[...] = jnp.zeros((G, H), jnp.float32)
        scnt[...] = jnp.zeros((G, H), jnp.float32)

    dinv = _dinv_of(d_ref)
    h = jnp.tanh(dinv * (_hcat(a_ref) + _hcat(u_ref)) + b_ref[...])

    bid_row = batch_ref[0, :, :]
    ids = lax.broadcasted_iota(jnp.int32, (G, _PB), 0)
    onehot = (bid_row == ids).astype(jnp.float32)
    ssum[...] += jnp.dot(onehot, h, preferred_element_type=jnp.float32)
    scnt[...] += jnp.dot(onehot, jnp.ones((_PB, H), jnp.float32),
                         preferred_element_type=jnp.float32)

    gids = lax.broadcasted_iota(jnp.int32, (G, 1), 0)

    def maxbody(g, _):
        m = bcol_ref[...] == g
        contrib = jnp.max(jnp.where(m, h, -jnp.inf), axis=0, keepdims=True)
        rowm = gids == g
        smax[...] = jnp.where(rowm, jnp.maximum(smax[...], contrib), smax[...])
        return 0

    lax.fori_loop(batch_ref[0, 0, 0], batch_ref[0, 0, _PB - 1] + 1, maxbody, 0)

    @pl.when(i == _NPB - 1)
    def _():
        gmax = jnp.where(scnt[...] > 0, smax[...], 0.0)
        gmean = ssum[...] / jnp.maximum(scnt[...], 1.0)
        res = jnp.sum(gmax * w_ref[0:1, :] + gmean * w_ref[1:2, :],
                      axis=1, keepdims=True) + bo_ref[:, 0:1]
        o_ref[...] = jnp.broadcast_to(res, (G, H))


def _tc_final(agg, u, deg2, b, batch, wcat, boutb):
    grid = (_NPB,)
    out = pl.pallas_call(
        _tc_final_body,
        grid=grid,
        in_specs=[
            pl.BlockSpec((NC, _PB, HD), lambda i: (0, i, 0)),
            pl.BlockSpec((NC, _PB, HD), lambda i: (0, i, 0)),
            pl.BlockSpec((NC, _PB, 16), lambda i: (0, i, 0)),
            pl.BlockSpec((1, H), lambda i: (0, 0)),
            pl.BlockSpec((1, 1, _PB), lambda i: (i, 0, 0)),
            pl.BlockSpec((_PB, 1), lambda i: (i, 0)),
            pl.BlockSpec((2, H), lambda i: (0, 0)),
            pl.BlockSpec((1, H), lambda i: (0, 0)),
        ],
        out_specs=pl.BlockSpec((G, H), lambda i: (0, 0)),
        out_shape=jax.ShapeDtypeStruct((G, H), jnp.float32),
        scratch_shapes=[
            pltpu.VMEM((G, H), jnp.float32),
            pltpu.VMEM((G, H), jnp.float32),
            pltpu.VMEM((G, H), jnp.float32),
        ],
    )(agg, u, deg2, b.reshape(1, H), batch.reshape(_NPB, 1, _PB),
      batch.reshape(N, 1), wcat, boutb)
    return out[:, 0:1]


def kernel(x, edge_index, batch, W0, b0, W1, b1, W2, b2, W3, b3, Wout, bout):
    src = edge_index[0:1]
    dst = edge_index[1:2]
    er = jnp.concatenate([src, src + N, dst], axis=0).reshape(3, NS, CHT, CW)
    wcat = Wout.reshape(2, H)
    boutb = jnp.broadcast_to(bout.reshape(1, 1), (1, H))

    deg2 = _sc_deg(er)
    u = _tc_layer0(x, deg2, W0)
    bs = [b0, b1, b2, b3]
    Ws = [W1, W2, W3]
    for l in range(4):
        agg = _sc_agg(u.reshape(2 * N, HD), er)
        if l < 3:
            u = _tc_layer(agg, u, deg2, bs[l], Ws[l])
    return _tc_final(agg, u, deg2, b3, batch, wcat, boutb)


# speedup vs baseline: 23.2523x; 1.7698x over previous
import functools

import jax
import jax.numpy as jnp
from jax import lax
from jax.experimental import pallas as pl
from jax.experimental.pallas import tpu as pltpu
from jax.experimental.pallas import tpu_sc as plsc

N = 10000
E = 320000
H = 128
HD = H // 2
G = 64

NC = 2
NS = 16
CW = 125
CHT = 160
RPT = 624
TAIL = N - NS * RPT
ZB = 208

_mesh = plsc.VectorSubcoreMesh(core_axis_name="c", subcore_axis_name="s")


@functools.partial(
    pl.kernel,
    out_type=jax.ShapeDtypeStruct((NC, N, 16), jnp.float32),
    mesh=_mesh,
    scratch_types=[
        pltpu.VMEM((CHT, CW), jnp.int32),
        pltpu.VMEM((CW, 16), jnp.float32),
        pltpu.VMEM((RPT, 16), jnp.float32),
        pltpu.VMEM_SHARED((N, 16), jnp.float32),
    ],
    compiler_params=pltpu.CompilerParams(use_tc_tiling_on_sc=False),
)
def _sc_deg(er_hbm, out_hbm, dst_v, ones_v, zero_v, acc_sh):
    c = lax.axis_index("c")
    s = lax.axis_index("s")

    @pl.loop(0, CW)
    def _(i):
        ones_v[i, :] = jnp.ones((16,), jnp.float32)

    @pl.loop(0, RPT)
    def _(i):
        zero_v[i, :] = jnp.zeros((16,), jnp.float32)

    pltpu.sync_copy(zero_v, acc_sh.at[pl.ds(s * RPT, RPT)])

    @pl.when(s == NS - 1)
    def _():
        pltpu.sync_copy(zero_v.at[pl.ds(0, TAIL)],
                        acc_sh.at[pl.ds(NS * RPT, TAIL)])

    plsc.subcore_barrier()

    pltpu.sync_copy(er_hbm.at[2, s], dst_v)

    @pl.loop(0, CHT // NC)
    def _(j):
        pltpu.sync_copy(ones_v, acc_sh.at[dst_v.at[c * (CHT // NC) + j]],
                        add=True)

    plsc.subcore_barrier()
    pltpu.sync_copy(acc_sh.at[pl.ds(s * RPT, RPT)],
                    out_hbm.at[c, pl.ds(s * RPT, RPT)])

    @pl.when(s == NS - 1)
    def _():
        pltpu.sync_copy(acc_sh.at[pl.ds(NS * RPT, TAIL)],
                        out_hbm.at[c, pl.ds(NS * RPT, TAIL)])


@functools.partial(
    pl.kernel,
    out_type=jax.ShapeDtypeStruct((NC, N, HD), jnp.float32),
    mesh=_mesh,
    scratch_types=[
        pltpu.VMEM((CHT, CW), jnp.int32),
        pltpu.VMEM((CHT, CW), jnp.int32),
        pltpu.VMEM((4, CW, HD), jnp.float32),
        pltpu.VMEM((ZB, HD), jnp.float32),
        pltpu.VMEM_SHARED((N, HD), jnp.float32),
        pltpu.SemaphoreType.DMA((4,)),
        pltpu.SemaphoreType.DMA((4,)),
    ],
    compiler_params=pltpu.CompilerParams(use_tc_tiling_on_sc=False),
)
def _sc_agg(u_flat, er_hbm, out_hbm, src_v, dst_v, rows_v, zero_v, acc_sh,
            gsem, ssem):
    u_hbm = u_flat
    c = lax.axis_index("c")
    s = lax.axis_index("s")

    @pl.loop(0, ZB)
    def _(i):
        for k in range(HD // 16):
            zero_v[i, pl.ds(16 * k, 16)] = jnp.zeros((16,), jnp.float32)

    for t in range(RPT // ZB):
        pltpu.sync_copy(zero_v, acc_sh.at[pl.ds(s * RPT + t * ZB, ZB)])

    @pl.when(s == NS - 1)
    def _():
        pltpu.sync_copy(zero_v.at[pl.ds(0, TAIL)],
                        acc_sh.at[pl.ds(NS * RPT, TAIL)])

    plsc.subcore_barrier()

    pltpu.sync_copy(er_hbm.at[c, s], src_v)
    pltpu.sync_copy(er_hbm.at[2, s], dst_v)

    def g_start(ch, b):
        pltpu.async_copy(u_hbm.at[src_v.at[ch]], rows_v.at[b], gsem.at[b])

    def g_wait(ch, b):
        pltpu.make_async_copy(u_hbm.at[src_v.at[ch]], rows_v.at[b],
                              gsem.at[b]).wait()

    def s_start(ch, b):
        pltpu.async_copy(rows_v.at[b], acc_sh.at[dst_v.at[ch]], ssem.at[b],
                         add=True)

    def s_wait(ch, b):
        pltpu.make_async_copy(rows_v.at[b], acc_sh.at[dst_v.at[ch]],
                              ssem.at[b]).wait()

    g_start(0, 0)
    g_start(1, 1)
    g_start(2, 2)

    @pl.loop(0, CHT, step=4)
    def _(j):
        for b in range(4):
            ch = j + b
            g_wait(ch, b)
            s_start(ch, b)
            nb = (b + 3) % 4

            @pl.when(ch + 3 < CHT)
            def _():
                @pl.when(ch >= 1)
                def _():
                    s_wait(ch - 1, nb)

                g_start(ch + 3, nb)

    for k in range(4):
        s_wait(CHT - 4 + k, k)

    plsc.subcore_barrier()
    pltpu.sync_copy(acc_sh.at[pl.ds(s * RPT, RPT)],
                    out_hbm.at[c, pl.ds(s * RPT, RPT)])

    @pl.when(s == NS - 1)
    def _():
        pltpu.sync_copy(acc_sh.at[pl.ds(NS * RPT, TAIL)],
                        out_hbm.at[c, pl.ds(NS * RPT, TAIL)])


_RB = 2000


def _dinv_of(dref):
    deg = dref[0, :, 0:1] + dref[1, :, 0:1] + 1.0
    return lax.rsqrt(deg)


def _split_store(o_ref, t, dinv):
    u = t * dinv
    o_ref[0:1, :, :] = u[None, :, 0:HD]
    o_ref[1:2, :, :] = u[None, :, HD:H]


def _tc_layer0_body(x_ref, d_ref, w_ref, o_ref):
    dinv = _dinv_of(d_ref)
    t = jnp.dot(x_ref[...], w_ref[...], preferred_element_type=jnp.float32)
    _split_store(o_ref, t, dinv)


def _tc_layer0(x, deg2, W0):
    grid = (N // _RB,)
    return pl.pallas_call(
        _tc_layer0_body,
        grid=grid,
        in_specs=[
            pl.BlockSpec((_RB, H), lambda i: (i, 0)),
            pl.BlockSpec((NC, _RB, 16), lambda i: (0, i, 0)),
            pl.BlockSpec((H, H), lambda i: (0, 0)),
        ],
        out_specs=pl.BlockSpec((NC, _RB, HD), lambda i: (0, i, 0)),
        out_shape=jax.ShapeDtypeStruct((NC, N, HD), jnp.float32),
    )(x, deg2, W0)


def _hcat(a_ref):
    return jnp.concatenate([a_ref[0], a_ref[1]], axis=1)


def _tc_layer_body(a_ref, u_ref, d_ref, b_ref, w_ref, o_ref):
    dinv = _dinv_of(d_ref)
    pre = dinv * (_hcat(a_ref) + _hcat(u_ref)) + b_ref[...]
    h = jnp.tanh(pre)
    t = jnp.dot(h, w_ref[...], preferred_element_type=jnp.float32)
    _split_store(o_ref, t, dinv)


def _tc_layer(agg, u, deg2, b, Wnext):
    grid = (N // _RB,)
    return pl.pallas_call(
        _tc_layer_body,
        grid=grid,
        in_specs=[
            pl.BlockSpec((NC, _RB, HD), lambda i: (0, i, 0)),
            pl.BlockSpec((NC, _RB, HD), lambda i: (0, i, 0)),
            pl.BlockSpec((NC, _RB, 16), lambda i: (0, i, 0)),
            pl.BlockSpec((1, H), lambda i: (0, 0)),
            pl.BlockSpec((H, H), lambda i: (0, 0)),
        ],
        out_specs=pl.BlockSpec((NC, _RB, HD), lambda i: (0, i, 0)),
        out_shape=jax.ShapeDtypeStruct((NC, N, HD), jnp.float32),
    )(agg, u, deg2, b.reshape(1, H), Wnext)


_PB = 400
_NPB = N // _PB


def _tc_final_body(a_ref, u_ref, d_ref, b_ref, batch_ref, bcol_ref, w_ref,
                   bo_ref, o_ref, smax, ssum, scnt):
    i = pl.program_id(0)

    @pl.when(i == 0)
    def _():
        smax[...] = jnp.full((G, H), -jnp.inf, jnp.float32)
        ssum[...] = jnp.zeros((G, H), jnp.float32)
        scnt[...] = jnp.zeros((G, H), jnp.float32)

    dinv = _dinv_of(d_ref)
    h = jnp.tanh(dinv * (_hcat(a_ref) + _hcat(u_ref)) + b_ref[...])

    bid_row = batch_ref[0, :, :]
    ids = lax.broadcasted_iota(jnp.int32, (G, _PB), 0)
    onehot = (bid_row == ids).astype(jnp.float32)
    ssum[...] += jnp.dot(onehot, h, preferred_element_type=jnp.float32)
    scnt[...] += jnp.dot(onehot, jnp.ones((_PB, H), jnp.float32),
                         preferred_element_type=jnp.float32)

    gids = lax.broadcasted_iota(jnp.int32, (G, 1), 0)

    def maxbody(g, _):
        m = bcol_ref[...] == g
        contrib = jnp.max(jnp.where(m, h, -jnp.inf), axis=0, keepdims=True)
        rowm = gids == g
        smax[...] = jnp.where(rowm, jnp.maximum(smax[...], contrib), smax[...])
        return 0

    lax.fori_loop(batch_ref[0, 0, 0], batch_ref[0, 0, _PB - 1] + 1, maxbody, 0)

    @pl.when(i == _NPB - 1)
    def _():
        gmax = jnp.where(scnt[...] > 0, smax[...], 0.0)
        gmean = ssum[...] / jnp.maximum(scnt[...], 1.0)
        res = jnp.sum(gmax * w_ref[0:1, :] + gmean * w_ref[1:2, :],
                      axis=1, keepdims=True) + bo_ref[:, 0:1]
        o_ref[...] = jnp.broadcast_to(res, (G, H))


def _tc_final(agg, u, deg2, b, batch, wcat, boutb):
    grid = (_NPB,)
    out = pl.pallas_call(
        _tc_final_body,
        grid=grid,
        in_specs=[
            pl.BlockSpec((NC, _PB, HD), lambda i: (0, i, 0)),
            pl.BlockSpec((NC, _PB, HD), lambda i: (0, i, 0)),
            pl.BlockSpec((NC, _PB, 16), lambda i: (0, i, 0)),
            pl.BlockSpec((1, H), lambda i: (0, 0)),
            pl.BlockSpec((1, 1, _PB), lambda i: (i, 0, 0)),
            pl.BlockSpec((_PB, 1), lambda i: (i, 0)),
            pl.BlockSpec((2, H), lambda i: (0, 0)),
            pl.BlockSpec((1, H), lambda i: (0, 0)),
        ],
        out_specs=pl.BlockSpec((G, H), lambda i: (0, 0)),
        out_shape=jax.ShapeDtypeStruct((G, H), jnp.float32),
        scratch_shapes=[
            pltpu.VMEM((G, H), jnp.float32),
            pltpu.VMEM((G, H), jnp.float32),
            pltpu.VMEM((G, H), jnp.float32),
        ],
    )(agg, u, deg2, b.reshape(1, H), batch.reshape(_NPB, 1, _PB),
      batch.reshape(N, 1), wcat, boutb)
    return out[:, 0:1]


def kernel(x, edge_index, batch, W0, b0, W1, b1, W2, b2, W3, b3, Wout, bout):
    src = edge_index[0:1]
    dst = edge_index[1:2]
    er = jnp.concatenate([src, src + N, dst], axis=0).reshape(3, NS, CHT, CW)
    wcat = Wout.reshape(2, H)
    boutb = jnp.broadcast_to(bout.reshape(1, 1), (1, H))

    deg2 = _sc_deg(er)
    u = _tc_layer0(x, deg2, W0)
    bs = [b0, b1, b2, b3]
    Ws = [W1, W2, W3]
    for l in range(4):
        agg = _sc_agg(u.reshape(2 * N, HD), er)
        if l < 3:
            u = _tc_layer(agg, u, deg2, bs[l], Ws[l])
    return _tc_final(agg, u, deg2, b3, batch, wcat, boutb)


# speedup vs baseline: 25.5561x; 1.0991x over previous
import functools

import jax
import jax.numpy as jnp
from jax import lax
from jax.experimental import pallas as pl
from jax.experimental.pallas import tpu as pltpu
from jax.experimental.pallas import tpu_sc as plsc

N = 10000
E = 320000
H = 128
HD = H // 2
G = 64

NC = 2
NS = 16
CW = 125
CHT = 160
RPT = 624
TAIL = N - NS * RPT
ZB = 208

_mesh = plsc.VectorSubcoreMesh(core_axis_name="c", subcore_axis_name="s")


@functools.partial(
    pl.kernel,
    out_type=jax.ShapeDtypeStruct((NC, N, 16), jnp.float32),
    mesh=_mesh,
    scratch_types=[
        pltpu.VMEM((CHT, CW), jnp.int32),
        pltpu.VMEM((CW, 16), jnp.float32),
        pltpu.VMEM((RPT, 16), jnp.float32),
        pltpu.VMEM_SHARED((N, 16), jnp.float32),
    ],
    compiler_params=pltpu.CompilerParams(use_tc_tiling_on_sc=False),
)
def _sc_deg(er_hbm, out_hbm, dst_v, ones_v, zero_v, acc_sh):
    c = lax.axis_index("c")
    s = lax.axis_index("s")

    @pl.loop(0, CW)
    def _(i):
        ones_v[i, :] = jnp.ones((16,), jnp.float32)

    @pl.loop(0, RPT)
    def _(i):
        zero_v[i, :] = jnp.zeros((16,), jnp.float32)

    pltpu.sync_copy(zero_v, acc_sh.at[pl.ds(s * RPT, RPT)])

    @pl.when(s == NS - 1)
    def _():
        pltpu.sync_copy(zero_v.at[pl.ds(0, TAIL)],
                        acc_sh.at[pl.ds(NS * RPT, TAIL)])

    plsc.subcore_barrier()

    pltpu.sync_copy(er_hbm.at[2, s], dst_v)

    @pl.loop(0, CHT // NC)
    def _(j):
        pltpu.sync_copy(ones_v, acc_sh.at[dst_v.at[c * (CHT // NC) + j]],
                        add=True)

    plsc.subcore_barrier()
    pltpu.sync_copy(acc_sh.at[pl.ds(s * RPT, RPT)],
                    out_hbm.at[c, pl.ds(s * RPT, RPT)])

    @pl.when(s == NS - 1)
    def _():
        pltpu.sync_copy(acc_sh.at[pl.ds(NS * RPT, TAIL)],
                        out_hbm.at[c, pl.ds(NS * RPT, TAIL)])


@functools.partial(
    pl.kernel,
    out_type=jax.ShapeDtypeStruct((NC, N, HD), jnp.float32),
    mesh=_mesh,
    scratch_types=[
        pltpu.VMEM((CHT, CW), jnp.int32),
        pltpu.VMEM((CHT, CW), jnp.int32),
        pltpu.VMEM((4, CW, HD), jnp.float32),
        pltpu.VMEM((ZB, HD), jnp.float32),
        pltpu.VMEM_SHARED((N, HD), jnp.float32),
        pltpu.SemaphoreType.DMA((4,)),
        pltpu.SemaphoreType.DMA((4,)),
    ],
    compiler_params=pltpu.CompilerParams(use_tc_tiling_on_sc=False),
)
def _sc_agg(u_hbm, er_hbm, out_hbm, src_v, dst_v, rows_v, zero_v, acc_sh,
            gsem, ssem):
    c = lax.axis_index("c")
    s = lax.axis_index("s")

    @pl.loop(0, ZB)
    def _(i):
        for k in range(HD // 16):
            zero_v[i, pl.ds(16 * k, 16)] = jnp.zeros((16,), jnp.float32)

    for t in range(RPT // ZB):
        pltpu.sync_copy(zero_v, acc_sh.at[pl.ds(s * RPT + t * ZB, ZB)])

    @pl.when(s == NS - 1)
    def _():
        pltpu.sync_copy(zero_v.at[pl.ds(0, TAIL)],
                        acc_sh.at[pl.ds(NS * RPT, TAIL)])

    plsc.subcore_barrier()

    pltpu.sync_copy(er_hbm.at[c, s], src_v)
    pltpu.sync_copy(er_hbm.at[2, s], dst_v)

    def g_start(ch, b):
        pltpu.async_copy(u_hbm.at[src_v.at[ch]], rows_v.at[b], gsem.at[b])

    def g_wait(ch, b):
        pltpu.make_async_copy(u_hbm.at[src_v.at[ch]], rows_v.at[b],
                              gsem.at[b]).wait()

    def s_start(ch, b):
        pltpu.async_copy(rows_v.at[b], acc_sh.at[dst_v.at[ch]], ssem.at[b],
                         add=True)

    def s_wait(ch, b):
        pltpu.make_async_copy(rows_v.at[b], acc_sh.at[dst_v.at[ch]],
                              ssem.at[b]).wait()

    g_start(0, 0)
    g_start(1, 1)
    g_start(2, 2)

    @pl.loop(0, CHT, step=4)
    def _(j):
        for b in range(4):
            ch = j + b
            g_wait(ch, b)
            s_start(ch, b)
            nb = (b + 3) % 4

            @pl.when(ch + 3 < CHT)
            def _():
                @pl.when(ch >= 1)
                def _():
                    s_wait(ch - 1, nb)

                g_start(ch + 3, nb)

    for k in range(4):
        s_wait(CHT - 4 + k, k)

    plsc.subcore_barrier()
    pltpu.sync_copy(acc_sh.at[pl.ds(s * RPT, RPT)],
                    out_hbm.at[c, pl.ds(s * RPT, RPT)])

    @pl.when(s == NS - 1)
    def _():
        pltpu.sync_copy(acc_sh.at[pl.ds(NS * RPT, TAIL)],
                        out_hbm.at[c, pl.ds(NS * RPT, TAIL)])


_RB = 2000


def _dinv_of(dref):
    deg = dref[0, :, 0:1] + dref[1, :, 0:1] + 1.0
    return lax.rsqrt(deg)


def _tc_layer0_body(x_ref, d_ref, w_ref, o_ref):
    dinv = _dinv_of(d_ref)
    t = jnp.dot(x_ref[...], w_ref[...], preferred_element_type=jnp.float32)
    o_ref[...] = t * dinv


def _tc_layer0(x, deg2, W0):
    grid = (N // _RB,)
    return pl.pallas_call(
        _tc_layer0_body,
        grid=grid,
        in_specs=[
            pl.BlockSpec((_RB, H), lambda i: (i, 0)),
            pl.BlockSpec((NC, _RB, 16), lambda i: (0, i, 0)),
            pl.BlockSpec((H, H), lambda i: (0, 0)),
        ],
        out_specs=pl.BlockSpec((_RB, H), lambda i: (i, 0)),
        out_shape=jax.ShapeDtypeStruct((N, H), jnp.float32),
    )(x, deg2, W0)


def _hcat(a_ref):
    return jnp.concatenate([a_ref[0], a_ref[1]], axis=1)


def _tc_layer_body(a_ref, u_ref, d_ref, b_ref, w_ref, o_ref):
    dinv = _dinv_of(d_ref)
    h = jnp.tanh(dinv * (_hcat(a_ref) + u_ref[...]) + b_ref[...])
    t = jnp.dot(h, w_ref[...], preferred_element_type=jnp.float32)
    o_ref[...] = t * dinv


def _tc_layer(agg, u, deg2, b, Wnext):
    grid = (N // _RB,)
    return pl.pallas_call(
        _tc_layer_body,
        grid=grid,
        in_specs=[
            pl.BlockSpec((NC, _RB, HD), lambda i: (0, i, 0)),
            pl.BlockSpec((_RB, H), lambda i: (i, 0)),
            pl.BlockSpec((NC, _RB, 16), lambda i: (0, i, 0)),
            pl.BlockSpec((1, H), lambda i: (0, 0)),
            pl.BlockSpec((H, H), lambda i: (0, 0)),
        ],
        out_specs=pl.BlockSpec((_RB, H), lambda i: (i, 0)),
        out_shape=jax.ShapeDtypeStruct((N, H), jnp.float32),
    )(agg, u, deg2, b.reshape(1, H), Wnext)


_PB = 1000
_NPB = N // _PB


def _tc_final_body(a_ref, u_ref, d_ref, b_ref, batch_ref, bcol_ref, w_ref,
                   bo_ref, o_ref, smax, ssum, scnt):
    i = pl.program_id(0)

    @pl.when(i == 0)
    def _():
        smax[...] = jnp.full((G, H), -jnp.inf, jnp.float32)
        ssum[...] = jnp.zeros((G, H), jnp.float32)
        scnt[...] = jnp.zeros((G, H), jnp.float32)

    dinv = _dinv_of(d_ref)
    h = jnp.tanh(dinv * (_hcat(a_ref) + u_ref[...]) + b_ref[...])

    bid_row = batch_ref[0, :, :]
    ids = lax.broadcasted_iota(jnp.int32, (G, _PB), 0)
    onehot = (bid_row == ids).astype(jnp.float32)
    ssum[...] += jnp.dot(onehot, h, preferred_element_type=jnp.float32)
    scnt[...] += jnp.dot(onehot, jnp.ones((_PB, H), jnp.float32),
                         preferred_element_type=jnp.float32)

    gids = lax.broadcasted_iota(jnp.int32, (G, 1), 0)

    def maxbody(g, _):
        m = bcol_ref[...] == g
        contrib = jnp.max(jnp.where(m, h, -jnp.inf), axis=0, keepdims=True)
        rowm = gids == g
        smax[...] = jnp.where(rowm, jnp.maximum(smax[...], contrib), smax[...])
        return 0

    lax.fori_loop(batch_ref[0, 0, 0], batch_ref[0, 0, _PB - 1] + 1, maxbody, 0)

    @pl.when(i == _NPB - 1)
    def _():
        gmax = jnp.where(scnt[...] > 0, smax[...], 0.0)
        gmean = ssum[...] / jnp.maximum(scnt[...], 1.0)
        res = jnp.sum(gmax * w_ref[0:1, :] + gmean * w_ref[1:2, :],
                      axis=1, keepdims=True) + bo_ref[:, 0:1]
        o_ref[...] = jnp.broadcast_to(res, (G, H))


def _tc_final(agg, u, deg2, b, batch, wcat, boutb):
    grid = (_NPB,)
    out = pl.pallas_call(
        _tc_final_body,
        grid=grid,
        in_specs=[
            pl.BlockSpec((NC, _PB, HD), lambda i: (0, i, 0)),
            pl.BlockSpec((_PB, H), lambda i: (i, 0)),
            pl.BlockSpec((NC, _PB, 16), lambda i: (0, i, 0)),
            pl.BlockSpec((1, H), lambda i: (0, 0)),
            pl.BlockSpec((1, 1, _PB), lambda i: (i, 0, 0)),
            pl.BlockSpec((_PB, 1), lambda i: (i, 0)),
            pl.BlockSpec((2, H), lambda i: (0, 0)),
            pl.BlockSpec((1, H), lambda i: (0, 0)),
        ],
        out_specs=pl.BlockSpec((G, H), lambda i: (0, 0)),
        out_shape=jax.ShapeDtypeStruct((G, H), jnp.float32),
        scratch_shapes=[
            pltpu.VMEM((G, H), jnp.float32),
            pltpu.VMEM((G, H), jnp.float32),
            pltpu.VMEM((G, H), jnp.float32),
        ],
    )(agg, u, deg2, b.reshape(1, H), batch.reshape(_NPB, 1, _PB),
      batch.reshape(N, 1), wcat, boutb)
    return out[:, 0:1]


def kernel(x, edge_index, batch, W0, b0, W1, b1, W2, b2, W3, b3, Wout, bout):
    src = edge_index[0:1]
    dst = edge_index[1:2]
    er = jnp.concatenate([2 * src, 2 * src + 1, dst],
                         axis=0).reshape(3, NS, CHT, CW)
    wcat = Wout.reshape(2, H)
    boutb = jnp.broadcast_to(bout.reshape(1, 1), (1, H))

    deg2 = _sc_deg(er)
    u = _tc_layer0(x, deg2, W0)
    bs = [b0, b1, b2, b3]
    Ws = [W1, W2, W3]
    for l in range(4):
        agg = _sc_agg(u.reshape(2 * N, HD), er)
        if l < 3:
            u = _tc_layer(agg, u, deg2, bs[l], Ws[l])
    return _tc_final(agg, u, deg2, b3, batch, wcat, boutb)
